# baseline (speedup 1.0000x reference)
import jax
import jax.numpy as jnp
from jax.experimental import pallas as pl

N = 10000
B = 64
ED = 128
SEQ = 1000


def _final_mm(xc, Wf1, bf1, Wf2, bf2, Wo, bo):
    def body(xc_ref, w1_ref, b1_ref, w2_ref, b2_ref, wo_ref, bo_ref, o_ref):
        h = jnp.maximum(xc_ref[...] @ w1_ref[...] + b1_ref[...], 0.0)
        h = jnp.maximum(h @ w2_ref[...] + b2_ref[...], 0.0)
        o_ref[...] = h @ wo_ref[...] + bo_ref[...]

    return pl.pallas_call(
        body,
        out_shape=jax.ShapeDtypeStruct((B, 1), jnp.float32),
    )(xc, Wf1, bf1[None, :], Wf2, bf2[None, :], Wo, bo[None, :])


def kernel(x, edge_index, batch, target, params):
    p = params
    counts = jnp.bincount(batch, length=B)
    v_freq = counts.max()
    starts = jnp.concatenate([jnp.zeros((1,), counts.dtype), jnp.cumsum(counts)[:-1]])
    pos = jnp.arange(N, dtype=counts.dtype) - starts[batch]

    loop = jnp.arange(N, dtype=edge_index.dtype)
    src = jnp.concatenate([edge_index[0], loop])
    dst = jnp.concatenate([edge_index[1], loop])
    deg = jnp.zeros((N,), jnp.float32).at[dst].add(1.0)
    dinv = jax.lax.rsqrt(deg)
    norm = dinv[src] * dinv[dst]

    def gcn(h, W, b):
        m = h @ W
        out = jnp.zeros((N, W.shape[1]), h.dtype).at[dst].add(m[src] * norm[:, None])
        return out + b

    h = jax.nn.relu(gcn(x, p['W1'], p['b1']))
    idn = h
    h = jax.nn.relu(gcn(h, p['W2'], p['b2']))
    h = h + jnp.pad(idn, ((0, 0), (0, h.shape[1] - idn.shape[1])))
    h = jax.nn.relu(gcn(h, p['W3'], p['b3']))
    idn = h
    h = jax.nn.relu(gcn(h, p['W4'], p['b4']))
    h = h + jnp.pad(idn, ((0, 0), (0, h.shape[1] - idn.shape[1])))
    h = jax.nn.relu(gcn(h, p['W5'], p['b5']))
    h = jax.nn.relu(gcn(h, p['W6'], p['b6']))

    emb_xt = jnp.take(p['emb'], target, axis=0)
    x_resh = jnp.zeros((B, N, ED), jnp.float32).at[batch, pos].set(h)
    g = jax.ops.segment_max(h, batch, num_segments=B)
    g = jax.nn.relu(g @ p['Wg1'] + p['bg1'])
    g = g @ p['Wg2'] + p['bg2']
    q = emb_xt @ p['Wattn_in']
    scores = jnp.einsum('bqd,bkd->bqk', q, x_resh)
    scores = jnp.where(jnp.arange(N) < v_freq, scores, -jnp.inf)
    w = jax.nn.softmax(scores, axis=-1)
    mix = jnp.einsum('bqk,bkd->bqd', w, x_resh)
    comb = jnp.concatenate([mix, emb_xt], axis=-1)
    attn_out = jnp.tanh(comb @ p['Wattn_out'])
    conv = jax.lax.conv_general_dilated(attn_out, p['Wconv'], (1,), 'VALID',
                                        dimension_numbers=('NCH', 'OIH', 'NCH'))
    conv = conv + p['bconv'][None, :, None]
    xt = conv.reshape(B, 32 * 121) @ p['Wxt'] + p['bxt']
    xc = jnp.concatenate([g, xt], axis=1)
    return _final_mm(xc, p['Wf1'], p['bf1'], p['Wf2'], p['bf2'], p['Wo'], p['bo'])


# speedup vs baseline: 12.8463x; 12.8463x over previous
import functools

import jax
import jax.numpy as jnp
from jax import lax
from jax.experimental import pallas as pl
from jax.experimental.pallas import tpu as pltpu
from jax.experimental.pallas import tpu_sc as plsc

N = 10000
E = 640000
B = 64
ED = 128
SEQ = 1000
D = 128

NW = 32
KCH = 128
NCH = 157
EPT = KCH * NCH
EPAD = NW * EPT - E
ACC = 10240
RPT = ACC // 16

KB = 256
RB = 1000


@functools.cache
def _make_sc_degree():
    mesh = plsc.VectorSubcoreMesh(core_axis_name="c", subcore_axis_name="s")

    @functools.partial(
        pl.kernel,
        mesh=mesh,
        out_type=jax.ShapeDtypeStruct((2, ACC), jnp.float32),
        scratch_types=[
            pltpu.VMEM((NCH, KCH), jnp.int32),
            pltpu.VMEM((KCH,), jnp.float32),
            pltpu.VMEM_SHARED((ACC,), jnp.float32),
        ],
    )
    def _sc_degree(dst_hbm, z1_hbm, out_hbm, dst_v, ones_v, acc_sh):
        cid = lax.axis_index("c")
        sid = lax.axis_index("s")
        wid = sid * 2 + cid

        def fill_ones(i, _):
            ones_v[pl.ds(i * 16, 16)] = jnp.ones((16,), jnp.float32)
            return 0

        lax.fori_loop(0, KCH // 16, fill_ones, 0)
        pltpu.sync_copy(z1_hbm, acc_sh.at[pl.ds(sid * RPT, RPT)])
        plsc.subcore_barrier()

        pltpu.sync_copy(dst_hbm.at[wid], dst_v)

        def step(j, _):
            pltpu.sync_copy(ones_v, acc_sh.at[dst_v.at[j]], add=True)
            return 0

        lax.fori_loop(0, NCH, step, 0)
        plsc.subcore_barrier()
        pltpu.sync_copy(acc_sh.at[pl.ds(sid * RPT, RPT)],
                        out_hbm.at[cid, pl.ds(sid * RPT, RPT)])

    return _sc_degree


@functools.cache
def _make_sc_scatter():
    mesh = plsc.VectorSubcoreMesh(core_axis_name="c", subcore_axis_name="s")

    @functools.partial(
        pl.kernel,
        mesh=mesh,
        out_type=jax.ShapeDtypeStruct((2, ACC, D), jnp.float32),
        scratch_types=[
            pltpu.VMEM((KCH,), jnp.int32),
            pltpu.VMEM((KCH,), jnp.int32),
            pltpu.VMEM((KCH, D), jnp.float32),
            pltpu.VMEM_SHARED((ACC, D), jnp.float32),
            pltpu.SemaphoreType.DMA,
        ],
    )
    def k(u_hbm, src_hbm, dst_hbm, z2_hbm, out_hbm, src_v, dst_v, rows_v,
          acc_sh, sem):
        cid = lax.axis_index("c")
        sid = lax.axis_index("s")
        wid = sid * 2 + cid

        pltpu.sync_copy(z2_hbm, acc_sh.at[pl.ds(sid * RPT, RPT)])
        plsc.subcore_barrier()

        def step(j, _):
            pltpu.sync_copy(src_hbm.at[wid, j], src_v)
            pltpu.sync_copy(dst_hbm.at[wid, j], dst_v)
            pltpu.async_copy(u_hbm.at[src_v], rows_v, sem).wait()
            pltpu.sync_copy(rows_v, acc_sh.at[dst_v], add=True)
            return 0

        lax.fori_loop(0, NCH, step, 0)
        plsc.subcore_barrier()
        pltpu.sync_copy(acc_sh.at[pl.ds(sid * RPT, RPT)],
                        out_hbm.at[cid, pl.ds(sid * RPT, RPT)])

    return k


def _dinv_of(degpt):
    deg = degpt[:, 0:1] + degpt[:, 1:2] + 1.0
    return lax.rsqrt(deg)


def _tc_prep(x_pad, degPT):
    def body(x_ref, d_ref, o_ref):
        o_ref[...] = x_ref[...] * _dinv_of(d_ref[...])

    return pl.pallas_call(
        body,
        grid=(N // RB,),
        in_specs=[
            pl.BlockSpec((RB, D), lambda r: (r, 0)),
            pl.BlockSpec((RB, 2), lambda r: (r, 0)),
        ],
        out_specs=pl.BlockSpec((RB, D), lambda r: (r, 0)),
        out_shape=jax.ShapeDtypeStruct((N, D), jnp.float32),
    )(x_pad, degPT)


def _tc_layer(P, u, degPT, W, b, r_in=None, emit_r=False, final=False):
    has_res = r_in is not None

    def body(*refs):
        if has_res:
            p_ref, u_ref, d_ref, r_ref, w_ref, b_ref = refs[:6]
            outs = refs[6:]
        else:
            p_ref, u_ref, d_ref, w_ref, b_ref = refs[:5]
            outs = refs[5:]
        dinv = _dinv_of(d_ref[...])
        t = (p_ref[0] + p_ref[1] + u_ref[...]) * dinv
        z = jnp.dot(t, w_ref[...], preferred_element_type=jnp.float32)
        r = jnp.maximum(z + b_ref[...], 0.0)
        if final:
            outs[0][...] = r
        elif emit_r:
            outs[0][...] = r * dinv
            outs[1][...] = r
        elif has_res:
            h = r + r_ref[...]
            outs[0][...] = h * dinv
        else:
            outs[0][...] = r * dinv

    in_specs = [
        pl.BlockSpec((2, RB, D), lambda r: (0, r, 0)),
        pl.BlockSpec((RB, D), lambda r: (r, 0)),
        pl.BlockSpec((RB, 2), lambda r: (r, 0)),
    ]
    args = [P, u, degPT]
    if has_res:
        in_specs.append(pl.BlockSpec((RB, D), lambda r: (r, 0)))
        args.append(r_in)
    in_specs += [
        pl.BlockSpec((D, D), lambda r: (0, 0)),
        pl.BlockSpec((1, D), lambda r: (0, 0)),
    ]
    args += [W, b]

    n_out = 2 if emit_r else 1
    out_specs = [pl.BlockSpec((RB, D), lambda r: (r, 0))] * n_out
    out_shape = [jax.ShapeDtypeStruct((N, D), jnp.float32)] * n_out
    if n_out == 1:
        out_specs, out_shape = out_specs[0], out_shape[0]

    return pl.pallas_call(
        body,
        grid=(N // RB,),
        in_specs=in_specs,
        out_specs=out_specs,
        out_shape=out_shape,
    )(*args)


def _tc_attention(counts, starts, vfreq, h_pad, target4, emb_pad, Wattn_in,
                  Wtop, Wbot, WconvR):
    def body(cnt_sm, st_sm, vf_sm, h_ref, t_ref, e_ref, wi_ref, wt_ref,
             wb_ref, wc_ref, z_ref, g_ref):
        bidx = pl.program_id(0)
        cnt = cnt_sm[bidx]
        start = st_sm[bidx]
        vf = vf_sm[0]

        tgt = t_ref[0]
        oh_iota = lax.broadcasted_iota(jnp.int32, (SEQ, 32), 1)
        O = (tgt == oh_iota).astype(jnp.float32)
        qemb = jnp.dot(e_ref[...], wi_ref[...],
                       preferred_element_type=jnp.float32)
        ebB = jnp.dot(e_ref[...], wb_ref[...],
                      preferred_element_type=jnp.float32)
        q = jnp.dot(O, qemb, preferred_element_type=jnp.float32)
        Eb = jnp.dot(O, ebB, preferred_element_type=jnp.float32)

        nkb = (cnt + KB - 1) // KB

        def kstep(kb, carry):
            M, Dn, Acc, gmax = carry
            base = start + kb * KB
            K = h_ref[pl.ds(base, KB), :]
            col = lax.broadcasted_iota(jnp.int32, (SEQ, KB), 1)
            valid = (col + kb * KB) < cnt
            s = lax.dot_general(q, K, (((1,), (1,)), ((), ())),
                                preferred_element_type=jnp.float32)
            s = jnp.where(valid, s, -1e30)
            m_new = jnp.maximum(M, jnp.max(s, axis=1, keepdims=True))
            Pmat = jnp.exp(s - m_new)
            alpha = jnp.exp(M - m_new)
            Dn = Dn * alpha + jnp.sum(Pmat, axis=1, keepdims=True)
            Acc = Acc * alpha + jnp.dot(Pmat, K,
                                        preferred_element_type=jnp.float32)
            rowi = lax.broadcasted_iota(jnp.int32, (KB, ED), 0)
            Km = jnp.where((rowi + kb * KB) < cnt, K, -jnp.inf)
            gmax = jnp.maximum(gmax, jnp.max(Km, axis=0, keepdims=True))
            return m_new, Dn, Acc, gmax

        M0 = jnp.full((SEQ, 1), -1e30, jnp.float32)
        D0 = jnp.zeros((SEQ, 1), jnp.float32)
        A0 = jnp.zeros((SEQ, ED), jnp.float32)
        g0 = jnp.full((1, ED), -jnp.inf, jnp.float32)
        M, Dn, Acc, gmax = lax.fori_loop(0, nkb, kstep, (M0, D0, A0, g0))

        padk = (vf - cnt).astype(jnp.float32)
        Mp = jnp.where(padk > 0, jnp.maximum(M, 0.0), M)
        sc = jnp.exp(M - Mp)
        Dn = Dn * sc + padk * jnp.exp(-jnp.maximum(Mp, 0.0))
        Acc = Acc * sc
        mix = Acc / Dn

        attn = jnp.tanh(jnp.dot(mix, wt_ref[...],
                                preferred_element_type=jnp.float32) + Eb)
        z_ref[0] = lax.dot_general(wc_ref[...], attn, (((0,), (0,)), ((), ())),
                                   preferred_element_type=jnp.float32)
        g_ref[0] = gmax

    grid_spec = pltpu.PrefetchScalarGridSpec(
        num_scalar_prefetch=3,
        grid=(B,),
        in_specs=[
            pl.BlockSpec((N + KB, ED), lambda b, *_: (0, 0)),
            pl.BlockSpec((1, SEQ, 1), lambda b, *_: (b, 0, 0)),
            pl.BlockSpec((32, ED), lambda b, *_: (0, 0)),
            pl.BlockSpec((ED, ED), lambda b, *_: (0, 0)),
            pl.BlockSpec((ED, ED), lambda b, *_: (0, 0)),
            pl.BlockSpec((ED, ED), lambda b, *_: (0, 0)),
            pl.BlockSpec((SEQ, 256), lambda b, *_: (0, 0)),
        ],
        out_specs=[
            pl.BlockSpec((1, 256, ED), lambda b, *_: (b, 0, 0)),
            pl.BlockSpec((1, 1, ED), lambda b, *_: (b, 0, 0)),
        ],
    )
    return pl.pallas_call(
        body,
        grid_spec=grid_spec,
        out_shape=[
            jax.ShapeDtypeStruct((B, 256, ED), jnp.float32),
            jax.ShapeDtypeStruct((B, 1, ED), jnp.float32),
        ],
    )(counts, starts, vfreq, h_pad, target4, emb_pad, Wattn_in, Wtop, Wbot,
      WconvR)


def _tc_final(Zr, g3, WxtAll, Wxtp, bconv_r, bxt_r, Wg1, bg1, Wg2, bg2,
              Wf1, bf1, Wf2, bf2, Wo, bo_r):
    def body(z_ref, g_ref, wxa_ref, wxp_ref, bc_ref, bxt_ref, wg1_ref,
             bg1_ref, wg2_ref, bg2_ref, wf1_ref, bf1_ref, wf2_ref, bf2_ref,
             wo_ref, bo_ref, o_ref):
        g = g_ref[:, 0, :]
        g1 = jnp.maximum(jnp.dot(g, wg1_ref[...],
                                 preferred_element_type=jnp.float32)
                         + bg1_ref[...], 0.0)
        g2 = jnp.dot(g1, wg2_ref[...],
                     preferred_element_type=jnp.float32) + bg2_ref[...]

        wsum = jnp.sum(wxp_ref[...], axis=1)
        bias = jnp.dot(bc_ref[...], wsum,
                       preferred_element_type=jnp.float32)
        xt = (jnp.dot(z_ref[...], wxa_ref[...],
                      preferred_element_type=jnp.float32) + bias
              + bxt_ref[...])

        xc = jnp.concatenate([g2, xt], axis=1)
        f1 = jnp.maximum(jnp.dot(xc, wf1_ref[...],
                                 preferred_element_type=jnp.float32)
                         + bf1_ref[...], 0.0)
        f2 = jnp.maximum(jnp.dot(f1, wf2_ref[...],
                                 preferred_element_type=jnp.float32)
                         + bf2_ref[...], 0.0)
        o_ref[...] = jnp.dot(f2, wo_ref[...],
                             preferred_element_type=jnp.float32) + bo_ref[...]

    return pl.pallas_call(
        body,
        out_shape=jax.ShapeDtypeStruct((B, 1), jnp.float32),
    )(Zr, g3, WxtAll, Wxtp, bconv_r, bxt_r, Wg1, bg1, Wg2, bg2, Wf1, bf1,
      Wf2, bf2, Wo, bo_r)


def kernel(x, edge_index, batch, target, params):
    p = params
    f32 = jnp.float32

    bnd = jnp.searchsorted(batch, jnp.arange(B + 1, dtype=jnp.int32)
                           ).astype(jnp.int32)
    starts = bnd[:B]
    counts = bnd[1:] - bnd[:B]
    vfreq = jnp.max(counts, keepdims=True)

    ar = jnp.arange(EPAD, dtype=jnp.int32)
    src_all = jnp.concatenate([edge_index[0], (ar * 131) % N])
    dst_all = jnp.concatenate([edge_index[1], N + (ar % 128)])
    src_r = src_all.reshape(NW, NCH, KCH)
    dst_r = dst_all.reshape(NW, NCH, KCH)

    z1 = jnp.zeros((RPT,), f32)
    z2 = jnp.zeros((RPT, D), f32)

    def wpad(W, bvec):
        Wp = jnp.zeros((D, D), f32).at[:W.shape[0], :W.shape[1]].set(W)
        bp = jnp.zeros((1, D), f32).at[0, :bvec.shape[0]].set(bvec)
        return Wp, bp

    Ws = [wpad(p['W%d' % (i + 1)], p['b%d' % (i + 1)]) for i in range(6)]

    x_pad = jnp.pad(x, ((0, 0), (0, D - x.shape[1])))

    degP = _make_sc_degree()(dst_r, z1)
    degPT = jnp.transpose(degP[:, :N])

    u = _tc_prep(x_pad, degPT)

    sc_scatter = _make_sc_scatter()
    r_saved = None
    h = None
    for i in range(6):
        P = sc_scatter(u, src_r, dst_r, z2)
        W, bvec = Ws[i]
        if i in (0, 2):
            u, r_saved = _tc_layer(P, u, degPT, W, bvec, emit_r=True)
        elif i in (1, 3):
            u = _tc_layer(P, u, degPT, W, bvec, r_in=r_saved)
        elif i == 4:
            u = _tc_layer(P, u, degPT, W, bvec)
        else:
            h = _tc_layer(P, u, degPT, W, bvec, final=True)

    h_pad = jnp.pad(h, ((0, KB), (0, 0)))
    target4 = target.astype(jnp.int32).reshape(B, SEQ, 1)
    emb_pad = jnp.zeros((32, ED), f32).at[:p['emb'].shape[0]].set(p['emb'])
    Wtop = p['Wattn_out'][:ED]
    Wbot = p['Wattn_out'][ED:]
    WconvR = jnp.transpose(p['Wconv'], (1, 2, 0)).reshape(SEQ, 256)

    Z, g3 = _tc_attention(counts, starts, vfreq, h_pad, target4, emb_pad,
                          p['Wattn_in'], Wtop, Wbot, WconvR)
    Zr = Z.reshape(B, 256 * ED)

    Wxt3 = p['Wxt'].reshape(32, 121, ED)
    Wxtp = jnp.pad(Wxt3, ((0, 0), (0, 7), (0, 0)))
    WxtAll = jnp.stack(
        [jnp.pad(Wxt3, ((0, 0), (t, 7 - t), (0, 0))) for t in range(8)],
        axis=0).reshape(8 * 32 * 128, ED)

    out = _tc_final(
        Zr, g3, WxtAll, Wxtp,
        p['bconv'].reshape(1, 32), p['bxt'].reshape(1, ED),
        p['Wg1'], p['bg1'].reshape(1, 1024),
        p['Wg2'], p['bg2'].reshape(1, ED),
        p['Wf1'], p['bf1'].reshape(1, 1024),
        p['Wf2'], p['bf2'].reshape(1, 512),
        p['Wo'], p['bo'].reshape(1, 1))
    return out


# speedup vs baseline: 22.4015x; 1.7438x over previous
import functools

import jax
import jax.numpy as jnp
from jax import lax
from jax.experimental import pallas as pl
from jax.experimental.pallas import tpu as pltpu
from jax.experimental.pallas import tpu_sc as plsc

N = 10000
E = 640000
B = 64
ED = 128
SEQ = 1000
D = 128

NW = 32
KCH = 128
NCH = 160
IG = 8
NG = NCH // IG
EPT = KCH * NCH
EPAD = NW * EPT - E
ACC = 10240
RPT = ACC // 16

KB = 256
RB = 1000


@functools.cache
def _make_sc_degree():
    mesh = plsc.VectorSubcoreMesh(core_axis_name="c", subcore_axis_name="s")

    @functools.partial(
        pl.kernel,
        mesh=mesh,
        out_type=jax.ShapeDtypeStruct((2, ACC), jnp.float32),
        scratch_types=[
            pltpu.VMEM((NCH, KCH), jnp.int32),
            pltpu.VMEM((KCH,), jnp.float32),
            pltpu.VMEM_SHARED((ACC,), jnp.float32),
        ],
    )
    def _sc_degree(dst_hbm, z1_hbm, out_hbm, dst_v, ones_v, acc_sh):
        cid = lax.axis_index("c")
        sid = lax.axis_index("s")
        wid = sid * 2 + cid

        def fill_ones(i, _):
            ones_v[pl.ds(i * 16, 16)] = jnp.ones((16,), jnp.float32)
            return 0

        lax.fori_loop(0, KCH // 16, fill_ones, 0)
        pltpu.sync_copy(z1_hbm, acc_sh.at[pl.ds(sid * RPT, RPT)])
        plsc.subcore_barrier()

        pltpu.sync_copy(dst_hbm.at[wid], dst_v)

        def step(j, _):
            pltpu.sync_copy(ones_v, acc_sh.at[dst_v.at[j]], add=True)
            return 0

        lax.fori_loop(0, NCH, step, 0)
        plsc.subcore_barrier()
        pltpu.sync_copy(acc_sh.at[pl.ds(sid * RPT, RPT)],
                        out_hbm.at[cid, pl.ds(sid * RPT, RPT)])

    return _sc_degree


@functools.cache
def _make_sc_scatter():
    mesh = plsc.VectorSubcoreMesh(core_axis_name="c", subcore_axis_name="s")

    @functools.partial(
        pl.kernel,
        mesh=mesh,
        out_type=jax.ShapeDtypeStruct((2, ACC, D), jnp.float32),
        scratch_types=[
            pltpu.VMEM((IG, KCH), jnp.int32),
            pltpu.VMEM((IG, KCH), jnp.int32),
            pltpu.VMEM((2, KCH, D), jnp.float32),
            pltpu.VMEM_SHARED((ACC, D), jnp.float32),
            pltpu.SemaphoreType.DMA,
            pltpu.SemaphoreType.DMA,
            pltpu.SemaphoreType.DMA,
            pltpu.SemaphoreType.DMA,
        ],
    )
    def k(u_hbm, src_hbm, dst_hbm, z2_hbm, out_hbm, src_v, dst_v, rows_v,
          acc_sh, semg0, semg1, sems0, sems1):
        cid = lax.axis_index("c")
        sid = lax.axis_index("s")
        wid = sid * 2 + cid
        semg = (semg0, semg1)
        sems = (sems0, sems1)

        pltpu.sync_copy(z2_hbm, acc_sh.at[pl.ds(sid * RPT, RPT)])
        plsc.subcore_barrier()

        def g_desc(kk, bb):
            return pltpu.make_async_copy(u_hbm.at[src_v.at[kk]],
                                         rows_v.at[bb], semg[bb])

        def s_desc(kk, bb):
            return pltpu.make_async_copy(rows_v.at[bb],
                                         acc_sh.at[dst_v.at[kk]], sems[bb])

        def group(ig, _):
            pltpu.sync_copy(src_hbm.at[wid, pl.ds(ig * IG, IG)], src_v)
            pltpu.sync_copy(dst_hbm.at[wid, pl.ds(ig * IG, IG)], dst_v)
            g_desc(0, 0).start()
            for kk in range(IG):
                bb = kk & 1
                if kk + 1 < IG:
                    if kk >= 1:
                        s_desc(kk - 1, 1 - bb).wait()
                    g_desc(kk + 1, 1 - bb).start()
                g_desc(kk, bb).wait()
                s_desc(kk, bb).start(add=True)
            s_desc(IG - 2, 0).wait()
            s_desc(IG - 1, 1).wait()
            return 0

        lax.fori_loop(0, NG, group, 0)
        plsc.subcore_barrier()
        pltpu.sync_copy(acc_sh.at[pl.ds(sid * RPT, RPT)],
                        out_hbm.at[cid, pl.ds(sid * RPT, RPT)])

    return k


def _dinv_of(degpt):
    deg = degpt[:, 0:1] + degpt[:, 1:2] + 1.0
    return lax.rsqrt(deg)


def _tc_prep(x_pad, degPT):
    def body(x_ref, d_ref, o_ref):
        o_ref[...] = x_ref[...] * _dinv_of(d_ref[...])

    return pl.pallas_call(
        body,
        grid=(N // RB,),
        in_specs=[
            pl.BlockSpec((RB, D), lambda r: (r, 0)),
            pl.BlockSpec((RB, 2), lambda r: (r, 0)),
        ],
        out_specs=pl.BlockSpec((RB, D), lambda r: (r, 0)),
        out_shape=jax.ShapeDtypeStruct((N, D), jnp.float32),
    )(x_pad, degPT)


def _tc_layer(P, u, degPT, W, b, r_in=None, emit_r=False, final=False):
    has_res = r_in is not None

    def body(*refs):
        if has_res:
            p_ref, u_ref, d_ref, r_ref, w_ref, b_ref = refs[:6]
            outs = refs[6:]
        else:
            p_ref, u_ref, d_ref, w_ref, b_ref = refs[:5]
            outs = refs[5:]
        dinv = _dinv_of(d_ref[...])
        t = (p_ref[0] + p_ref[1] + u_ref[...]) * dinv
        z = jnp.dot(t, w_ref[...], preferred_element_type=jnp.float32)
        r = jnp.maximum(z + b_ref[...], 0.0)
        if final:
            outs[0][...] = r
        elif emit_r:
            outs[0][...] = r * dinv
            outs[1][...] = r
        elif has_res:
            h = r + r_ref[...]
            outs[0][...] = h * dinv
        else:
            outs[0][...] = r * dinv

    in_specs = [
        pl.BlockSpec((2, RB, D), lambda r: (0, r, 0)),
        pl.BlockSpec((RB, D), lambda r: (r, 0)),
        pl.BlockSpec((RB, 2), lambda r: (r, 0)),
    ]
    args = [P, u, degPT]
    if has_res:
        in_specs.append(pl.BlockSpec((RB, D), lambda r: (r, 0)))
        args.append(r_in)
    in_specs += [
        pl.BlockSpec((D, D), lambda r: (0, 0)),
        pl.BlockSpec((1, D), lambda r: (0, 0)),
    ]
    args += [W, b]

    n_out = 2 if emit_r else 1
    out_specs = [pl.BlockSpec((RB, D), lambda r: (r, 0))] * n_out
    out_shape = [jax.ShapeDtypeStruct((N, D), jnp.float32)] * n_out
    if n_out == 1:
        out_specs, out_shape = out_specs[0], out_shape[0]

    return pl.pallas_call(
        body,
        grid=(N // RB,),
        in_specs=in_specs,
        out_specs=out_specs,
        out_shape=out_shape,
    )(*args)


def _tc_attention(counts, starts, vfreq, h_pad, target4, emb_pad, Wattn_in,
                  Wtop, Wbot, WconvR):
    def body(cnt_sm, st_sm, vf_sm, h_ref, t_ref, e_ref, wi_ref, wt_ref,
             wb_ref, wc_ref, z_ref, g_ref):
        bidx = pl.program_id(0)
        cnt = cnt_sm[bidx]
        start = st_sm[bidx]
        vf = vf_sm[0]

        tgt = t_ref[0]
        oh_iota = lax.broadcasted_iota(jnp.int32, (SEQ, 32), 1)
        O = (tgt == oh_iota).astype(jnp.float32)
        qemb = jnp.dot(e_ref[...], wi_ref[...],
                       preferred_element_type=jnp.float32)
        ebB = jnp.dot(e_ref[...], wb_ref[...],
                      preferred_element_type=jnp.float32)
        q = jnp.dot(O, qemb, preferred_element_type=jnp.float32)
        Eb = jnp.dot(O, ebB, preferred_element_type=jnp.float32)

        nkb = (cnt + KB - 1) // KB

        def kstep(kb, carry):
            M, Dn, Acc, gmax = carry
            base = start + kb * KB
            K = h_ref[pl.ds(base, KB), :]
            col = lax.broadcasted_iota(jnp.int32, (SEQ, KB), 1)
            valid = (col + kb * KB) < cnt
            s = lax.dot_general(q, K, (((1,), (1,)), ((), ())),
                                preferred_element_type=jnp.float32)
            s = jnp.where(valid, s, -1e30)
            m_new = jnp.maximum(M, jnp.max(s, axis=1, keepdims=True))
            Pmat = jnp.exp(s - m_new)
            alpha = jnp.exp(M - m_new)
            Dn = Dn * alpha + jnp.sum(Pmat, axis=1, keepdims=True)
            Acc = Acc * alpha + jnp.dot(Pmat, K,
                                        preferred_element_type=jnp.float32)
            rowi = lax.broadcasted_iota(jnp.int32, (KB, ED), 0)
            Km = jnp.where((rowi + kb * KB) < cnt, K, -jnp.inf)
            gmax = jnp.maximum(gmax, jnp.max(Km, axis=0, keepdims=True))
            return m_new, Dn, Acc, gmax

        M0 = jnp.full((SEQ, 1), -1e30, jnp.float32)
        D0 = jnp.zeros((SEQ, 1), jnp.float32)
        A0 = jnp.zeros((SEQ, ED), jnp.float32)
        g0 = jnp.full((1, ED), -jnp.inf, jnp.float32)
        M, Dn, Acc, gmax = lax.fori_loop(0, nkb, kstep, (M0, D0, A0, g0))

        padk = (vf - cnt).astype(jnp.float32)
        Mp = jnp.where(padk > 0, jnp.maximum(M, 0.0), M)
        sc = jnp.exp(M - Mp)
        Dn = Dn * sc + padk * jnp.exp(-jnp.maximum(Mp, 0.0))
        Acc = Acc * sc
        mix = Acc / Dn

        attn = jnp.tanh(jnp.dot(mix, wt_ref[...],
                                preferred_element_type=jnp.float32) + Eb)
        z_ref[0] = lax.dot_general(wc_ref[...], attn, (((0,), (0,)), ((), ())),
                                   preferred_element_type=jnp.float32)
        g_ref[0] = gmax

    grid_spec = pltpu.PrefetchScalarGridSpec(
        num_scalar_prefetch=3,
        grid=(B,),
        in_specs=[
            pl.BlockSpec((N + KB, ED), lambda b, *_: (0, 0)),
            pl.BlockSpec((1, SEQ, 1), lambda b, *_: (b, 0, 0)),
            pl.BlockSpec((32, ED), lambda b, *_: (0, 0)),
            pl.BlockSpec((ED, ED), lambda b, *_: (0, 0)),
            pl.BlockSpec((ED, ED), lambda b, *_: (0, 0)),
            pl.BlockSpec((ED, ED), lambda b, *_: (0, 0)),
            pl.BlockSpec((SEQ, 256), lambda b, *_: (0, 0)),
        ],
        out_specs=[
            pl.BlockSpec((1, 256, ED), lambda b, *_: (b, 0, 0)),
            pl.BlockSpec((1, 1, ED), lambda b, *_: (b, 0, 0)),
        ],
    )
    return pl.pallas_call(
        body,
        grid_spec=grid_spec,
        out_shape=[
            jax.ShapeDtypeStruct((B, 256, ED), jnp.float32),
            jax.ShapeDtypeStruct((B, 1, ED), jnp.float32),
        ],
    )(counts, starts, vfreq, h_pad, target4, emb_pad, Wattn_in, Wtop, Wbot,
      WconvR)


def _tc_final(Zr, g3, WxtAll, Wxtp, bconv_r, bxt_r, Wg1, bg1, Wg2, bg2,
              Wf1, bf1, Wf2, bf2, Wo, bo_r):
    def body(z_ref, g_ref, wxa_ref, wxp_ref, bc_ref, bxt_ref, wg1_ref,
             bg1_ref, wg2_ref, bg2_ref, wf1_ref, bf1_ref, wf2_ref, bf2_ref,
             wo_ref, bo_ref, o_ref):
        g = g_ref[:, 0, :]
        g1 = jnp.maximum(jnp.dot(g, wg1_ref[...],
                                 preferred_element_type=jnp.float32)
                         + bg1_ref[...], 0.0)
        g2 = jnp.dot(g1, wg2_ref[...],
                     preferred_element_type=jnp.float32) + bg2_ref[...]

        wsum = jnp.sum(wxp_ref[...], axis=1)
        bias = jnp.dot(bc_ref[...], wsum,
                       preferred_element_type=jnp.float32)
        xt = (jnp.dot(z_ref[...], wxa_ref[...],
                      preferred_element_type=jnp.float32) + bias
              + bxt_ref[...])

        xc = jnp.concatenate([g2, xt], axis=1)
        f1 = jnp.maximum(jnp.dot(xc, wf1_ref[...],
                                 preferred_element_type=jnp.float32)
                         + bf1_ref[...], 0.0)
        f2 = jnp.maximum(jnp.dot(f1, wf2_ref[...],
                                 preferred_element_type=jnp.float32)
                         + bf2_ref[...], 0.0)
        o_ref[...] = jnp.dot(f2, wo_ref[...],
                             preferred_element_type=jnp.float32) + bo_ref[...]

    return pl.pallas_call(
        body,
        out_shape=jax.ShapeDtypeStruct((B, 1), jnp.float32),
    )(Zr, g3, WxtAll, Wxtp, bconv_r, bxt_r, Wg1, bg1, Wg2, bg2, Wf1, bf1,
      Wf2, bf2, Wo, bo_r)


def kernel(x, edge_index, batch, target, params):
    p = params
    f32 = jnp.float32

    bnd = jnp.searchsorted(batch, jnp.arange(B + 1, dtype=jnp.int32)
                           ).astype(jnp.int32)
    starts = bnd[:B]
    counts = bnd[1:] - bnd[:B]
    vfreq = jnp.max(counts, keepdims=True)

    ar = jnp.arange(EPAD, dtype=jnp.int32)
    src_all = jnp.concatenate([edge_index[0], (ar * 131) % N])
    dst_all = jnp.concatenate([edge_index[1], N + (ar % 128)])
    src_r = src_all.reshape(NW, NCH, KCH)
    dst_r = dst_all.reshape(NW, NCH, KCH)

    z1 = jnp.zeros((RPT,), f32)
    z2 = jnp.zeros((RPT, D), f32)

    def wpad(W, bvec):
        Wp = jnp.zeros((D, D), f32).at[:W.shape[0], :W.shape[1]].set(W)
        bp = jnp.zeros((1, D), f32).at[0, :bvec.shape[0]].set(bvec)
        return Wp, bp

    Ws = [wpad(p['W%d' % (i + 1)], p['b%d' % (i + 1)]) for i in range(6)]

    x_pad = jnp.pad(x, ((0, 0), (0, D - x.shape[1])))

    degP = _make_sc_degree()(dst_r, z1)
    degPT = jnp.transpose(degP[:, :N])

    u = _tc_prep(x_pad, degPT)

    sc_scatter = _make_sc_scatter()
    r_saved = None
    h = None
    for i in range(6):
        P = sc_scatter(u, src_r, dst_r, z2)
        W, bvec = Ws[i]
        if i in (0, 2):
            u, r_saved = _tc_layer(P, u, degPT, W, bvec, emit_r=True)
        elif i in (1, 3):
            u = _tc_layer(P, u, degPT, W, bvec, r_in=r_saved)
        elif i == 4:
            u = _tc_layer(P, u, degPT, W, bvec)
        else:
            h = _tc_layer(P, u, degPT, W, bvec, final=True)

    h_pad = jnp.pad(h, ((0, KB), (0, 0)))
    target4 = target.astype(jnp.int32).reshape(B, SEQ, 1)
    emb_pad = jnp.zeros((32, ED), f32).at[:p['emb'].shape[0]].set(p['emb'])
    Wtop = p['Wattn_out'][:ED]
    Wbot = p['Wattn_out'][ED:]
    WconvR = jnp.transpose(p['Wconv'], (1, 2, 0)).reshape(SEQ, 256)

    Z, g3 = _tc_attention(counts, starts, vfreq, h_pad, target4, emb_pad,
                          p['Wattn_in'], Wtop, Wbot, WconvR)
    Zr = Z.reshape(B, 256 * ED)

    Wxt3 = p['Wxt'].reshape(32, 121, ED)
    Wxtp = jnp.pad(Wxt3, ((0, 0), (0, 7), (0, 0)))
    WxtAll = jnp.stack(
        [jnp.pad(Wxt3, ((0, 0), (t, 7 - t), (0, 0))) for t in range(8)],
        axis=0).reshape(8 * 32 * 128, ED)

    out = _tc_final(
        Zr, g3, WxtAll, Wxtp,
        p['bconv'].reshape(1, 32), p['bxt'].reshape(1, ED),
        p['Wg1'], p['bg1'].reshape(1, 1024),
        p['Wg2'], p['bg2'].reshape(1, ED),
        p['Wf1'], p['bf1'].reshape(1, 1024),
        p['Wf2'], p['bf2'].reshape(1, 512),
        p['Wo'], p['bo'].reshape(1, 1))
    return out
